# initial kernel scaffold (unmeasured)
import jax
import jax.numpy as jnp
from jax import lax
from jax.experimental import pallas as pl
from jax.experimental.pallas import tpu as pltpu

N_DEV = 8


def kernel(x, w_mat, scale_x, scale_w):
    m, k = x.shape
    n = w_mat.shape[1]
    n_per = n // N_DEV

    def body(x_ref, w_ref, sx_ref, sw_ref, out_ref, blk_ref, send_sems, recv_sems):
        my = lax.axis_index("i")
        scale = sx_ref[0] * sw_ref[0]

        barrier_sem = pltpu.get_barrier_semaphore()
        for kk in range(1, N_DEV):
            pl.semaphore_signal(
                barrier_sem, inc=1,
                device_id=((my + kk) % N_DEV,),
                device_id_type=pl.DeviceIdType.MESH,
            )
        pl.semaphore_wait(barrier_sem, N_DEV - 1)

        for kk in range(1, N_DEV):
            j = (my + kk) % N_DEV
            blk_ref[kk - 1] = (
                jnp.dot(x_ref[...], w_ref[:, pl.ds(j * n_per, n_per)],
                        preferred_element_type=jnp.float32)
                * scale
            )
            rdma = pltpu.make_async_remote_copy(
                src_ref=blk_ref.at[kk - 1],
                dst_ref=out_ref.at[pl.ds(my * m, m), :],
                send_sem=send_sems.at[kk - 1],
                recv_sem=recv_sems.at[kk - 1],
                device_id=(j,),
                device_id_type=pl.DeviceIdType.MESH,
            )
            rdma.start()

        out_ref[pl.ds(my * m, m), :] = (
            jnp.dot(x_ref[...], w_ref[:, pl.ds(my * n_per, n_per)],
                    preferred_element_type=jnp.float32)
            * scale
        )

        for kk in range(1, N_DEV):
            src = (my - kk) % N_DEV
            recv = pltpu.make_async_remote_copy(
                src_ref=blk_ref.at[kk - 1],
                dst_ref=out_ref.at[pl.ds(src * m, m), :],
                send_sem=send_sems.at[kk - 1],
                recv_sem=recv_sems.at[kk - 1],
                device_id=(src,),
                device_id_type=pl.DeviceIdType.MESH,
            )
            recv.wait_recv()
        for kk in range(1, N_DEV):
            send_wait = pltpu.make_async_remote_copy(
                src_ref=blk_ref.at[kk - 1],
                dst_ref=out_ref.at[pl.ds(my * m, m), :],
                send_sem=send_sems.at[kk - 1],
                recv_sem=recv_sems.at[kk - 1],
                device_id=((my + kk) % N_DEV,),
                device_id_type=pl.DeviceIdType.MESH,
            )
            send_wait.wait_send()

    return pl.pallas_call(
        body,
        out_shape=jax.ShapeDtypeStruct((N_DEV * m, n_per), jnp.float32),
        in_specs=[
            pl.BlockSpec(memory_space=pltpu.VMEM),
            pl.BlockSpec(memory_space=pltpu.VMEM),
            pl.BlockSpec(memory_space=pltpu.SMEM),
            pl.BlockSpec(memory_space=pltpu.SMEM),
        ],
        out_specs=pl.BlockSpec(memory_space=pltpu.VMEM),
        scratch_shapes=[
            pltpu.VMEM((N_DEV - 1, m, n_per), jnp.float32),
            pltpu.SemaphoreType.DMA((N_DEV - 1,)),
            pltpu.SemaphoreType.DMA((N_DEV - 1,)),
        ],
        compiler_params=pltpu.CompilerParams(collective_id=0),
    )(x, w_mat, scale_x, scale_w)


# baseline (device time: 143740 ns/iter reference)
import jax
import jax.numpy as jnp
from jax import lax
from jax.experimental import pallas as pl
from jax.experimental.pallas import tpu as pltpu

N_DEV = 8
NCOL = 512
N_SLOTS = 2 * (N_DEV - 1)


def kernel(x, w_mat, scale_x, scale_w):
    m, k = x.shape
    n = w_mat.shape[1]
    n_per = n // N_DEV
    xb = x.astype(jnp.bfloat16)

    def gemm_body(xb_ref, w_ref, sx_ref, sw_ref, y_ref, wbuf, wsems):
        scale = sx_ref[0] * sw_ref[0]

        def w_copy(t):
            return pltpu.make_async_copy(
                w_ref.at[:, pl.ds(t * NCOL, NCOL)],
                wbuf.at[t % 2],
                wsems.at[t % 2],
            )

        n_panels = n // NCOL
        w_copy(0).start()
        for t in range(n_panels):
            if t + 1 < n_panels:
                w_copy(t + 1).start()
            w_copy(t).wait()
            acc = jnp.dot(
                xb_ref[...], wbuf[t % 2].astype(jnp.bfloat16),
                preferred_element_type=jnp.float32,
            ) * scale
            y_ref[:, pl.ds(t * NCOL, NCOL)] = acc.astype(jnp.bfloat16)

    y = pl.pallas_call(
        gemm_body,
        out_shape=jax.ShapeDtypeStruct((m, n), jnp.bfloat16),
        in_specs=[
            pl.BlockSpec(memory_space=pltpu.VMEM),
            pl.BlockSpec(memory_space=pltpu.MemorySpace.HBM),
            pl.BlockSpec(memory_space=pltpu.SMEM),
            pl.BlockSpec(memory_space=pltpu.SMEM),
        ],
        out_specs=pl.BlockSpec(memory_space=pltpu.VMEM),
        scratch_shapes=[
            pltpu.VMEM((2, k, NCOL), jnp.float32),
            pltpu.SemaphoreType.DMA((2,)),
        ],
    )(xb, w_mat, scale_x, scale_w)

    def a2a_body(y_ref, out_ref, snd, rcv, send_sems, recv_sems):
        my = lax.axis_index("i")
        barrier_sem = pltpu.get_barrier_semaphore()
        for kk in range(1, N_DEV):
            pl.semaphore_signal(
                barrier_sem, inc=1,
                device_id=((my + kk) % N_DEV,),
                device_id_type=pl.DeviceIdType.MESH,
            )
        pl.semaphore_wait(barrier_sem, N_DEV - 1)

        def desc(slot, dev):
            return pltpu.make_async_remote_copy(
                src_ref=snd.at[slot], dst_ref=rcv.at[slot],
                send_sem=send_sems.at[slot], recv_sem=recv_sems.at[slot],
                device_id=(dev,), device_id_type=pl.DeviceIdType.MESH,
            )

        for t in range(N_SLOTS):
            j = (my + t // 2 + 1) % N_DEV
            col = j * n_per + (t % 2) * NCOL
            snd[t] = y_ref[:, pl.ds(col, NCOL)]
            desc(t, j).start()

        out_ref[pl.ds(my * m, m), :] = (
            y_ref[:, pl.ds(my * n_per, n_per)].astype(jnp.float32)
        )

        for slot in range(N_SLOTS):
            src = (my - slot // 2 - 1) % N_DEV
            desc(slot, src).wait_recv()
            out_ref[pl.ds(src * m, m), pl.ds((slot % 2) * NCOL, NCOL)] = (
                rcv[slot].astype(jnp.float32)
            )
        for slot in range(N_SLOTS):
            desc(slot, (my + slot // 2 + 1) % N_DEV).wait_send()

    return pl.pallas_call(
        a2a_body,
        out_shape=jax.ShapeDtypeStruct((N_DEV * m, n_per), jnp.float32),
        in_specs=[pl.BlockSpec(memory_space=pltpu.VMEM)],
        out_specs=pl.BlockSpec(memory_space=pltpu.VMEM),
        scratch_shapes=[
            pltpu.VMEM((N_SLOTS, m, NCOL), jnp.bfloat16),
            pltpu.VMEM((N_SLOTS, m, NCOL), jnp.bfloat16),
            pltpu.SemaphoreType.DMA((N_SLOTS,)),
            pltpu.SemaphoreType.DMA((N_SLOTS,)),
        ],
        compiler_params=pltpu.CompilerParams(collective_id=0),
    )(y)


# device time: 138649 ns/iter; 1.0367x vs baseline; 1.0367x over previous
import jax
import jax.numpy as jnp
from jax import lax
from jax.experimental import pallas as pl
from jax.experimental.pallas import tpu as pltpu

N_DEV = 8
NCOL = 512
N_SLOTS = 2 * (N_DEV - 1)


def kernel(x, w_mat, scale_x, scale_w):
    m, k = x.shape
    n = w_mat.shape[1]
    n_per = n // N_DEV
    w_panel = 512
    n_panels = n // w_panel

    def gemm_body(x_ref, w_ref, sx_ref, sw_ref, y_ref, xq_ref):
        t = pl.program_id(0)

        @pl.when(t == 0)
        def _():
            xq_ref[...] = x_ref[...].astype(jnp.float8_e5m2)

        scale = sx_ref[0] * sw_ref[0]
        acc = jnp.dot(
            xq_ref[...], w_ref[...].astype(jnp.float8_e5m2),
            preferred_element_type=jnp.float32,
        ) * scale
        y_ref[...] = acc.astype(jnp.bfloat16)

    y = pl.pallas_call(
        gemm_body,
        grid=(n_panels,),
        in_specs=[
            pl.BlockSpec((m, k), lambda t: (0, 0)),
            pl.BlockSpec((k, w_panel), lambda t: (0, t)),
            pl.BlockSpec((1,), lambda t: (0,), memory_space=pltpu.MemorySpace.SMEM),
            pl.BlockSpec((1,), lambda t: (0,), memory_space=pltpu.MemorySpace.SMEM),
        ],
        out_specs=pl.BlockSpec((m, w_panel), lambda t: (0, t)),
        out_shape=jax.ShapeDtypeStruct((m, n), jnp.bfloat16),
        scratch_shapes=[pltpu.VMEM((m, k), jnp.float8_e5m2)],
        compiler_params=pltpu.CompilerParams(
            vmem_limit_bytes=56 * 1024 * 1024,
        ),
    )(x, w_mat, scale_x, scale_w)

    def a2a_body(y_ref, out_ref, rcv, stg, osems, send_sems, recv_sems):
        my = lax.axis_index("i")
        barrier_sem = pltpu.get_barrier_semaphore()
        for kk in range(1, N_DEV):
            pl.semaphore_signal(
                barrier_sem, inc=1,
                device_id=((my + kk) % N_DEV,),
                device_id_type=pl.DeviceIdType.MESH,
            )
        pl.semaphore_wait(barrier_sem, N_DEV - 1)

        def desc(slot, dev, col):
            return pltpu.make_async_remote_copy(
                src_ref=y_ref.at[:, pl.ds(col, NCOL)],
                dst_ref=rcv.at[slot],
                send_sem=send_sems.at[slot], recv_sem=recv_sems.at[slot],
                device_id=(dev,), device_id_type=pl.DeviceIdType.MESH,
            )

        def out_copy(slot, row, col):
            return pltpu.make_async_copy(
                stg.at[slot],
                out_ref.at[pl.ds(row, m), pl.ds(col, NCOL)],
                osems.at[slot],
            )

        for t in range(N_SLOTS):
            j = (my + t // 2 + 1) % N_DEV
            desc(t, j, j * n_per + (t % 2) * NCOL).start()

        for h in range(2):
            stg[N_SLOTS + h] = (
                y_ref[:, pl.ds(my * n_per + h * NCOL, NCOL)].astype(jnp.float32)
            )
            out_copy(N_SLOTS + h, my * m, h * NCOL).start()

        for slot in range(N_SLOTS):
            src = (my - slot // 2 - 1) % N_DEV
            desc(slot, src, 0).wait_recv()
            stg[slot] = rcv[slot].astype(jnp.float32)
            out_copy(slot, src * m, (slot % 2) * NCOL).start()

        for slot in range(N_SLOTS):
            src = (my - slot // 2 - 1) % N_DEV
            out_copy(slot, src * m, (slot % 2) * NCOL).wait()
        for h in range(2):
            out_copy(N_SLOTS + h, my * m, h * NCOL).wait()
        for slot in range(N_SLOTS):
            j = (my + slot // 2 + 1) % N_DEV
            desc(slot, j, j * n_per + (slot % 2) * NCOL).wait_send()

    return pl.pallas_call(
        a2a_body,
        out_shape=jax.ShapeDtypeStruct((N_DEV * m, n_per), jnp.float32),
        in_specs=[pl.BlockSpec(memory_space=pltpu.MemorySpace.VMEM)],
        out_specs=pl.BlockSpec(memory_space=pltpu.MemorySpace.HBM),
        scratch_shapes=[
            pltpu.VMEM((N_SLOTS, m, NCOL), jnp.bfloat16),
            pltpu.VMEM((N_SLOTS + 2, m, NCOL), jnp.float32),
            pltpu.SemaphoreType.DMA((N_SLOTS + 2,)),
            pltpu.SemaphoreType.DMA((N_SLOTS,)),
            pltpu.SemaphoreType.DMA((N_SLOTS,)),
        ],
        compiler_params=pltpu.CompilerParams(collective_id=0),
    )(y)
